# baseline (speedup 1.0000x reference)
import functools

import jax
import jax.numpy as jnp
from jax import lax
from jax.experimental import pallas as pl
from jax.experimental.pallas import tpu as pltpu
from jax.experimental.pallas import tpu_sc as plsc

_NT = 25
_WIND_W = 4
_WIND_IDX = 10
_WSPM_MEAN = 2.5
_WSPM_SCALE = 1.8
_MAX_DELAY = 24.0
_N = 128
_F = 32
_CI = 64


def _sc_body(xs_hbm, xw_hbm, adj_hbm, dist_hbm, out_hbm,
             xs_v, wind_v, invs_v, adj_v, dist_v, trow_v, w0row_v, w1row_v,
             out_v, tr_v, xs_sem):
    nc = 2
    b = lax.axis_index("s") * nc + lax.axis_index("c")

    xs_cp = pltpu.make_async_copy(xs_hbm.at[b], xs_v, xs_sem)
    xs_cp.start()

    pltpu.sync_copy(xw_hbm.at[b], wind_v)
    for k in range(_N // 16):
        sl = pl.ds(16 * k, 16)
        acc = jnp.zeros((16,), jnp.float32)
        for t in range(_WIND_W):
            acc = acc + wind_v[t, sl]
        wspm = jnp.maximum(acc * (1.0 / _WIND_W) * _WSPM_SCALE + _WSPM_MEAN, 0.0)
        invs_v[sl] = 1.0 / (wspm * 3.6 + 0.001)

    iota = lax.iota(jnp.int32, 16)
    first = True
    for c in range(_N // _CI):
        pltpu.sync_copy(adj_hbm.at[b, pl.ds(c * _CI, _CI)], adj_v)
        pltpu.sync_copy(dist_hbm.at[pl.ds(c * _CI, _CI)], dist_v)
        if first:
            xs_cp.wait()
            first = False

        def row_body(ii, _):
            for k in range(_N // 16):
                sl = pl.ds(16 * k, 16)
                tau = jnp.minimum(dist_v[ii, sl] * invs_v[sl], _MAX_DELAY)
                tq = (_NT - 1.0) - tau
                t0i = jnp.minimum(tq.astype(jnp.int32), _NT - 2)
                w1 = tq - t0i.astype(jnp.float32)
                a = adj_v[ii, sl]
                w1a = a * w1
                trow_v[sl] = t0i * (_N * _F) + (iota + 16 * k) * _F
                w0row_v[sl] = a - w1a
                w1row_v[sl] = w1a

            z = jnp.zeros((16,), jnp.float32)
            for f in range(_F):
                tr_v[pl.ds(16 * f, 16)] = z

            def blk_body(k, _):
                sl = pl.ds(16 * k, 16)
                ib0 = trow_v[sl]
                w0 = w0row_v[sl]
                w1 = w1row_v[sl]
                ib1 = ib0 + (_N * _F)
                for f in range(_F):
                    y0 = plsc.load_gather(xs_v, [ib0 + f] if f else [ib0])
                    y1 = plsc.load_gather(xs_v, [ib1 + f] if f else [ib1])
                    plsc.addupdate(tr_v.at[pl.ds(16 * f, 16)], w0 * y0 + w1 * y1)
                return 0

            lax.fori_loop(0, _N // 16, blk_body, 0)
            ro = (c * _CI + ii) * _F
            for g in range(2):
                idxg = (iota + 16 * g) * 16
                s = plsc.load_gather(tr_v, [idxg])
                for l in range(1, 16):
                    s = s + plsc.load_gather(tr_v, [idxg + l])
                out_v[pl.ds(ro + 16 * g, 16)] = s
            return 0

        lax.fori_loop(0, _CI, row_body, 0)

    pltpu.sync_copy(out_v, out_hbm.at[b])


def kernel(x_raw, adj, dist_km):
    B, T, N, F = x_raw.shape
    assert (B, N, F) == (32, _N, _F)
    xs = lax.slice_in_dim(x_raw, T - _NT, T, axis=1)
    xs = xs.reshape(B, _NT * N * F)
    xw = x_raw[:, T - _WIND_W:, :, _WIND_IDX]
    mesh = plsc.VectorSubcoreMesh(core_axis_name="c", subcore_axis_name="s")
    run = functools.partial(
        pl.kernel,
        out_type=jax.ShapeDtypeStruct((B, N * F), jnp.float32),
        mesh=mesh,
        compiler_params=pltpu.CompilerParams(use_tc_tiling_on_sc=False, needs_layout_passes=False),
        scratch_types=[
            pltpu.VMEM((_NT * _N * _F,), jnp.float32),
            pltpu.VMEM((_WIND_W, _N), jnp.float32),
            pltpu.VMEM((_N,), jnp.float32),
            pltpu.VMEM((_CI, _N), jnp.float32),
            pltpu.VMEM((_CI, _N), jnp.float32),
            pltpu.VMEM((_N,), jnp.int32),
            pltpu.VMEM((_N,), jnp.float32),
            pltpu.VMEM((_N,), jnp.float32),
            pltpu.VMEM((_N * _F,), jnp.float32),
            pltpu.VMEM((_F * 16,), jnp.float32),
            pltpu.SemaphoreType.DMA,
        ],
    )(_sc_body)
    out = run(xs, xw, adj, dist_km)
    return out.reshape(B, N, F)


# speedup vs baseline: 7.6635x; 7.6635x over previous
import functools

import jax
import jax.numpy as jnp
from jax import lax
from jax.experimental import pallas as pl
from jax.experimental.pallas import tpu as pltpu
from jax.experimental.pallas import tpu_sc as plsc

_NT = 25
_WIND_W = 4
_WIND_IDX = 10
_WSPM_MEAN = 2.5
_WSPM_SCALE = 1.8
_MAX_DELAY = 24.0
_N = 128
_F = 32
_BT = 16
_NB = 32 - _BT
_NSB = 32 // _NB
_RS = _N // _NSB


def _speed_rows(wind):
    wspm = jnp.clip(jnp.mean(wind, axis=0) * _WSPM_SCALE + _WSPM_MEAN, 0.0, None)
    return wspm * 3.6 + 0.001


def _tc_body(xs_ref, adj_ref, dist_ref, out_ref):
    wind = xs_ref[0, _NT - _WIND_W:, :, _WIND_IDX]
    speed = _speed_rows(wind)
    tau = jnp.clip(dist_ref[...] / speed[None, :], 0.0, _MAX_DELAY)
    tq = (_NT - 1.0) - tau
    adj = adj_ref[0]
    acc = jnp.zeros((_N, _F), dtype=jnp.float32)
    for t in range(_NT):
        w = adj * jnp.maximum(0.0, 1.0 - jnp.abs(t - tq))
        acc = acc + jnp.dot(w, xs_ref[0, t], preferred_element_type=jnp.float32)
    out_ref[0] = acc


def _sc_body(xs_hbm, xw_hbm, adj_hbm, dist_hbm, out_hbm,
             xs_v, wind_v, invs_v, adj_v, dist_v, trow_v, w0row_v, w1row_v,
             out_v, xs_sem):
    wid = lax.axis_index("s") * 2 + lax.axis_index("c")
    b = wid // _NSB
    r0 = (wid % _NSB) * _RS

    xs_cp = pltpu.make_async_copy(xs_hbm.at[b], xs_v, xs_sem)
    xs_cp.start()

    pltpu.sync_copy(xw_hbm.at[b], wind_v)
    for k in range(_N // 16):
        sl = pl.ds(16 * k, 16)
        acc = jnp.zeros((16,), jnp.float32)
        for t in range(_WIND_W):
            acc = acc + wind_v[t, sl]
        wspm = jnp.maximum(acc * (1.0 / _WIND_W) * _WSPM_SCALE + _WSPM_MEAN, 0.0)
        invs_v[sl] = 1.0 / (wspm * 3.6 + 0.001)

    pltpu.sync_copy(adj_hbm.at[b, pl.ds(r0, _RS)], adj_v)
    pltpu.sync_copy(dist_hbm.at[pl.ds(r0, _RS)], dist_v)
    xs_cp.wait()

    iota = lax.iota(jnp.int32, 16)

    def row_body(ii, _):
        for k in range(_N // 16):
            sl = pl.ds(16 * k, 16)
            tau = jnp.minimum(dist_v[ii, sl] * invs_v[sl], _MAX_DELAY)
            tq = (_NT - 1.0) - tau
            t0i = jnp.minimum(tq.astype(jnp.int32), _NT - 2)
            w1 = tq - t0i.astype(jnp.float32)
            a = adj_v[ii, sl]
            w1a = a * w1
            trow_v[sl] = t0i * (_N * _F)
            w0row_v[sl] = a - w1a
            w1row_v[sl] = w1a

        def blk_body(k, carry):
            acc0, acc1 = carry
            sl = pl.ds(16 * k, 16)
            t0blk = trow_v[sl]
            w0blk = w0row_v[sl]
            w1blk = w1row_v[sl]
            kbase = (16 * k) * _F
            for jj in range(16):
                lane = jnp.full((16,), jj, jnp.int32)
                t0b = jnp.take(t0blk, lane)
                w0b = jnp.take(w0blk, lane)
                w1b = jnp.take(w1blk, lane)
                i00 = t0b + (kbase + jj * _F) + iota
                i01 = i00 + 16
                i10 = i00 + (_N * _F)
                i11 = i01 + (_N * _F)
                x00 = plsc.load_gather(xs_v, [i00])
                x01 = plsc.load_gather(xs_v, [i01])
                x10 = plsc.load_gather(xs_v, [i10])
                x11 = plsc.load_gather(xs_v, [i11])
                acc0 = acc0 + w0b * x00 + w1b * x10
                acc1 = acc1 + w0b * x01 + w1b * x11
            return acc0, acc1

        z = jnp.zeros((16,), jnp.float32)
        acc0, acc1 = lax.fori_loop(0, _N // 16, blk_body, (z, z))
        ro = ii * _F
        out_v[pl.ds(ro, 16)] = acc0
        out_v[pl.ds(ro + 16, 16)] = acc1
        return 0

    lax.fori_loop(0, _RS, row_body, 0)
    pltpu.sync_copy(out_v, out_hbm.at[b, pl.ds(r0 * _F, _RS * _F)])


def kernel(x_raw, adj, dist_km):
    B, T, N, F = x_raw.shape
    assert (B, N, F) == (32, _N, _F)
    xs4 = lax.slice_in_dim(x_raw, T - _NT, T, axis=1)

    out_tc = pl.pallas_call(
        _tc_body,
        grid=(_BT,),
        in_specs=[
            pl.BlockSpec((1, _NT, N, F), lambda b: (b, 0, 0, 0)),
            pl.BlockSpec((1, N, N), lambda b: (b, 0, 0)),
            pl.BlockSpec((N, N), lambda b: (0, 0)),
        ],
        out_specs=pl.BlockSpec((1, N, F), lambda b: (b, 0, 0)),
        out_shape=jax.ShapeDtypeStruct((_BT, N, F), jnp.float32),
    )(lax.slice_in_dim(xs4, 0, _BT, axis=0), lax.slice_in_dim(adj, 0, _BT, axis=0), dist_km)

    xs_sc = lax.slice_in_dim(xs4, _BT, B, axis=0).reshape(_NB, _NT * N * F)
    xw_sc = x_raw[_BT:, T - _WIND_W:, :, _WIND_IDX]
    adj_sc = lax.slice_in_dim(adj, _BT, B, axis=0)
    mesh = plsc.VectorSubcoreMesh(core_axis_name="c", subcore_axis_name="s")
    run = functools.partial(
        pl.kernel,
        out_type=jax.ShapeDtypeStruct((_NB, N * F), jnp.float32),
        mesh=mesh,
        compiler_params=pltpu.CompilerParams(use_tc_tiling_on_sc=False,
                                             needs_layout_passes=False),
        scratch_types=[
            pltpu.VMEM((_NT * _N * _F,), jnp.float32),
            pltpu.VMEM((_WIND_W, _N), jnp.float32),
            pltpu.VMEM((_N,), jnp.float32),
            pltpu.VMEM((_RS, _N), jnp.float32),
            pltpu.VMEM((_RS, _N), jnp.float32),
            pltpu.VMEM((_N,), jnp.int32),
            pltpu.VMEM((_N,), jnp.float32),
            pltpu.VMEM((_N,), jnp.float32),
            pltpu.VMEM((_RS * _F,), jnp.float32),
            pltpu.SemaphoreType.DMA,
        ],
    )(_sc_body)
    out_sc = run(xs_sc, xw_sc, adj_sc, dist_km)
    return jnp.concatenate([out_tc, out_sc.reshape(_NB, N, F)], axis=0)


# speedup vs baseline: 8.4052x; 1.0968x over previous
import functools

import jax
import jax.numpy as jnp
from jax import lax
from jax.experimental import pallas as pl
from jax.experimental.pallas import tpu as pltpu
from jax.experimental.pallas import tpu_sc as plsc

_NT = 25
_WIND_W = 4
_WIND_IDX = 10
_WSPM_MEAN = 2.5
_WSPM_SCALE = 1.8
_MAX_DELAY = 24.0
_N = 128
_F = 32
_BT = 24
_NB = 32 - _BT
_NSB = 32 // _NB
_RS = _N // _NSB


def _speed_rows(wind):
    wspm = jnp.clip(jnp.mean(wind, axis=0) * _WSPM_SCALE + _WSPM_MEAN, 0.0, None)
    return wspm * 3.6 + 0.001


def _tc_body(xs_ref, adj_ref, dist_ref, out_ref):
    wind = xs_ref[0, _NT - _WIND_W:, :, _WIND_IDX]
    speed = _speed_rows(wind)
    tau = jnp.clip(dist_ref[...] / speed[None, :], 0.0, _MAX_DELAY)
    tq = (_NT - 1.0) - tau
    adj = adj_ref[0]
    acc = jnp.zeros((_N, _F), dtype=jnp.float32)
    for t in range(_NT):
        w = adj * jnp.maximum(0.0, 1.0 - jnp.abs(t - tq))
        acc = acc + jnp.dot(w, xs_ref[0, t], preferred_element_type=jnp.float32)
    out_ref[0] = acc


def _sc_body(xs_hbm, xw_hbm, adj_hbm, dist_hbm, out_hbm,
             xs_v, wind_v, invs_v, adj_v, dist_v, trow_v, w0row_v, w1row_v,
             out_v, xs_sem):
    wid = lax.axis_index("s") * 2 + lax.axis_index("c")
    b = wid // _NSB
    r0 = (wid % _NSB) * _RS

    xs_cp = pltpu.make_async_copy(xs_hbm.at[b], xs_v, xs_sem)
    xs_cp.start()

    pltpu.sync_copy(xw_hbm.at[b], wind_v)
    for k in range(_N // 16):
        sl = pl.ds(16 * k, 16)
        acc = jnp.zeros((16,), jnp.float32)
        for t in range(_WIND_W):
            acc = acc + wind_v[t, sl]
        wspm = jnp.maximum(acc * (1.0 / _WIND_W) * _WSPM_SCALE + _WSPM_MEAN, 0.0)
        invs_v[sl] = 1.0 / (wspm * 3.6 + 0.001)

    pltpu.sync_copy(adj_hbm.at[b, pl.ds(r0, _RS)], adj_v)
    pltpu.sync_copy(dist_hbm.at[pl.ds(r0, _RS)], dist_v)
    xs_cp.wait()

    iota = lax.iota(jnp.int32, 16)

    def row_body(ii, _):
        for k in range(_N // 16):
            sl = pl.ds(16 * k, 16)
            tau = jnp.minimum(dist_v[ii, sl] * invs_v[sl], _MAX_DELAY)
            tq = (_NT - 1.0) - tau
            t0i = jnp.minimum(tq.astype(jnp.int32), _NT - 2)
            w1 = tq - t0i.astype(jnp.float32)
            a = adj_v[ii, sl]
            w1a = a * w1
            trow_v[sl] = t0i * (_N * _F)
            w0row_v[sl] = a - w1a
            w1row_v[sl] = w1a

        def blk_body(k, carry):
            acc0, acc1 = carry
            sl = pl.ds(16 * k, 16)
            t0blk = trow_v[sl]
            w0blk = w0row_v[sl]
            w1blk = w1row_v[sl]
            kbase = (16 * k) * _F
            for jj in range(16):
                lane = jnp.full((16,), jj, jnp.int32)
                t0b = jnp.take(t0blk, lane)
                w0b = jnp.take(w0blk, lane)
                w1b = jnp.take(w1blk, lane)
                i00 = t0b + (kbase + jj * _F) + iota
                i01 = i00 + 16
                i10 = i00 + (_N * _F)
                i11 = i01 + (_N * _F)
                x00 = plsc.load_gather(xs_v, [i00])
                x01 = plsc.load_gather(xs_v, [i01])
                x10 = plsc.load_gather(xs_v, [i10])
                x11 = plsc.load_gather(xs_v, [i11])
                acc0 = acc0 + w0b * x00 + w1b * x10
                acc1 = acc1 + w0b * x01 + w1b * x11
            return acc0, acc1

        z = jnp.zeros((16,), jnp.float32)
        acc0, acc1 = lax.fori_loop(0, _N // 16, blk_body, (z, z))
        ro = ii * _F
        out_v[pl.ds(ro, 16)] = acc0
        out_v[pl.ds(ro + 16, 16)] = acc1
        return 0

    lax.fori_loop(0, _RS, row_body, 0)
    pltpu.sync_copy(out_v, out_hbm.at[b, pl.ds(r0 * _F, _RS * _F)])


def kernel(x_raw, adj, dist_km):
    B, T, N, F = x_raw.shape
    assert (B, N, F) == (32, _N, _F)
    xs4 = lax.slice_in_dim(x_raw, T - _NT, T, axis=1)

    out_tc = pl.pallas_call(
        _tc_body,
        grid=(_BT,),
        in_specs=[
            pl.BlockSpec((1, _NT, N, F), lambda b: (b, 0, 0, 0)),
            pl.BlockSpec((1, N, N), lambda b: (b, 0, 0)),
            pl.BlockSpec((N, N), lambda b: (0, 0)),
        ],
        out_specs=pl.BlockSpec((1, N, F), lambda b: (b, 0, 0)),
        out_shape=jax.ShapeDtypeStruct((_BT, N, F), jnp.float32),
    )(lax.slice_in_dim(xs4, 0, _BT, axis=0), lax.slice_in_dim(adj, 0, _BT, axis=0), dist_km)

    xs_sc = lax.slice_in_dim(xs4, _BT, B, axis=0).reshape(_NB, _NT * N * F)
    xw_sc = x_raw[_BT:, T - _WIND_W:, :, _WIND_IDX]
    adj_sc = lax.slice_in_dim(adj, _BT, B, axis=0)
    mesh = plsc.VectorSubcoreMesh(core_axis_name="c", subcore_axis_name="s")
    run = functools.partial(
        pl.kernel,
        out_type=jax.ShapeDtypeStruct((_NB, N * F), jnp.float32),
        mesh=mesh,
        compiler_params=pltpu.CompilerParams(use_tc_tiling_on_sc=False,
                                             needs_layout_passes=False),
        scratch_types=[
            pltpu.VMEM((_NT * _N * _F,), jnp.float32),
            pltpu.VMEM((_WIND_W, _N), jnp.float32),
            pltpu.VMEM((_N,), jnp.float32),
            pltpu.VMEM((_RS, _N), jnp.float32),
            pltpu.VMEM((_RS, _N), jnp.float32),
            pltpu.VMEM((_N,), jnp.int32),
            pltpu.VMEM((_N,), jnp.float32),
            pltpu.VMEM((_N,), jnp.float32),
            pltpu.VMEM((_RS * _F,), jnp.float32),
            pltpu.SemaphoreType.DMA,
        ],
    )(_sc_body)
    out_sc = run(xs_sc, xw_sc, adj_sc, dist_km)
    return jnp.concatenate([out_tc, out_sc.reshape(_NB, N, F)], axis=0)
